# Initial kernel scaffold; baseline (speedup 1.0000x reference)
import jax
import jax.numpy as jnp
from jax.experimental import pallas as pl


def kernel(x):
    raise NotImplementedError("write your pallas kernel here")


# speedup vs baseline: 1.2492x; 1.2492x over previous
import jax
import jax.numpy as jnp
from jax import lax
from jax.experimental import pallas as pl

_KTOP = 64
_ALPHA = 6.26
_NEG_INF = float("-inf")


def _body(x_ref, o_ref):
    w = x_ref[...]
    r, d = w.shape
    iota = lax.broadcasted_iota(jnp.int32, (r, d), 1).astype(jnp.float32)

    wp0 = jnp.maximum(w, 0.0)
    wn0 = jnp.maximum(-w, 0.0)
    s_pos = jnp.sum(wp0, axis=1, keepdims=True)
    s_neg = jnp.sum(wn0, axis=1, keepdims=True)

    def make_step(acc_is_index):
        def step(_, carry):
            arr, acc = carry
            m = jnp.max(arr, axis=1, keepdims=True)
            cand = jnp.where(arr == m, iota, jnp.float32(1e9))
            amin = jnp.min(cand, axis=1, keepdims=True)
            sel = cand == amin
            arr = jnp.where(sel, _NEG_INF, arr)
            acc = acc + (amin if acc_is_index else m)
            return arr, acc

        return step

    zero_col = jnp.zeros((r, 1), jnp.float32)
    k = _KTOP // 2
    wp_f, sum_idx = lax.fori_loop(0, k, make_step(True), (wp0, zero_col))
    wn_f, sum_nv = lax.fori_loop(0, k, make_step(False), (wn0, zero_col))

    pos_tmp = _ALPHA * (s_pos - sum_idx)
    neg_tmp = _ALPHA * (s_neg - sum_nv)

    pos_mask = wp_f == _NEG_INF
    neg_mask = wn_f == _NEG_INF
    out = jnp.where(pos_mask, iota + pos_tmp, 0.0)
    out = out + jnp.where(neg_mask, w - neg_tmp, 0.0)
    o_ref[...] = out


def kernel(x):
    b, d = x.shape
    rows = 8
    return pl.pallas_call(
        _body,
        grid=(b // rows,),
        in_specs=[pl.BlockSpec((rows, d), lambda i: (i, 0))],
        out_specs=pl.BlockSpec((rows, d), lambda i: (i, 0)),
        out_shape=jax.ShapeDtypeStruct((b, d), jnp.float32),
    )(x)


# speedup vs baseline: 9.7029x; 7.7670x over previous
import jax
import jax.numpy as jnp
from jax import lax
from jax.experimental import pallas as pl
from jax.experimental.pallas import tpu as pltpu
from jax.experimental.pallas import tpu_sc as plsc

_ALPHA = 6.26
_D = 32768
_B = 128
_L = 16
_GROUPS = 128
_VPG = 16
_NSUB = 32
_ROWS_PER_W = _B // _NSUB


def _iota16():
    return lax.broadcasted_iota(jnp.int32, (_L,), 0)


def _store1(ref, pos, val, i16):
    idx = jnp.broadcast_to(pos, (_L,)).astype(jnp.int32)
    v = jnp.broadcast_to(val, (_L,))
    plsc.store_scatter(ref, [idx], v, mask=i16 == 0)


def _sc_body(x_hbm, o_hbm, row_v, out_v, maxbuf, minbuf,
             posidx, negidx, negval):
    i16 = _iota16()
    wid = lax.axis_index("c") * 16 + lax.axis_index("s")

    def zero_step(i, _):
        out_v[pl.ds(i * _L, _L)] = jnp.zeros((_L,), jnp.float32)
        return 0

    lax.fori_loop(0, _D // _L, zero_step, 0)

    for j in range(_ROWS_PER_W):
        row = wid * _ROWS_PER_W + j
        pltpu.sync_copy(x_hbm.at[row], row_v)

        def p1_group(g, carry):
            accp, accn, m_all, mn_all = carry
            gmx = jnp.full((_L,), -jnp.inf, jnp.float32)
            gmn = jnp.full((_L,), jnp.inf, jnp.float32)

            def p1_vreg(k, c):
                gmx, gmn, accp, accn = c
                v = row_v[pl.ds(g * 256 + k * _L, _L)]
                return (jnp.maximum(gmx, v), jnp.minimum(gmn, v),
                        accp + jnp.maximum(v, 0.0), accn + jnp.minimum(v, 0.0))

            gmx, gmn, accp, accn = lax.fori_loop(
                0, _VPG, p1_vreg, (gmx, gmn, accp, accn))
            plsc.store_scatter(maxbuf, [i16 * _GROUPS + g], gmx)
            plsc.store_scatter(minbuf, [i16 * _GROUPS + g], gmn)
            return (accp, accn, jnp.maximum(m_all, gmx), jnp.minimum(mn_all, gmn))

        zv = jnp.zeros((_L,), jnp.float32)
        accp, accn, m_all, mn_all = lax.fori_loop(
            0, _GROUPS, p1_group,
            (zv, zv, jnp.full((_L,), -jnp.inf, jnp.float32),
             jnp.full((_L,), jnp.inf, jnp.float32)))
        s_pos = jnp.sum(accp)
        s_negsum = jnp.sum(accn)

        def extract(buf, m_vec, idx_list, val_list, is_max):
            def step(i, carry):
                m_vec, acc = carry
                m = jnp.max(m_vec) if is_max else jnp.min(m_vec)
                l = jnp.min(jnp.where(m_vec == m, i16, _L))
                gv = []
                for t in range(_GROUPS // _L):
                    gvec = buf[pl.ds(l * _GROUPS + t * _L, _L)]
                    gv.append(jnp.where(gvec == m, i16 + t * _L, _GROUPS))
                gmin = gv[0]
                for t in range(1, _GROUPS // _L):
                    gmin = jnp.minimum(gmin, gv[t])
                g = jnp.min(gmin)
                base = g * 256 + l
                idx = base + i16 * _L
                v = plsc.load_gather(row_v, [idx])
                kl = jnp.min(jnp.where(v == m, i16, _L))
                elem = base + kl * _L
                v2 = jnp.where(i16 == kl, 0.0, v)
                plsc.store_scatter(row_v, [idx], v2)
                newm = jnp.max(v2) if is_max else jnp.min(v2)
                _store1(buf, l * _GROUPS + g, newm, i16)
                _store1(idx_list, i, elem, i16)
                if val_list is not None:
                    _store1(val_list, i, m, i16)
                lv = []
                for t in range(_GROUPS // _L):
                    lv.append(buf[pl.ds(l * _GROUPS + t * _L, _L)])
                lacc = lv[0]
                for t in range(1, _GROUPS // _L):
                    lacc = jnp.maximum(lacc, lv[t]) if is_max \
                        else jnp.minimum(lacc, lv[t])
                ml = jnp.max(lacc) if is_max else jnp.min(lacc)
                m_vec = jnp.where(i16 == l, ml, m_vec)
                acc = acc + (elem.astype(jnp.float32) if is_max else m)
                return (m_vec, acc)

            return lax.fori_loop(0, 32, step, (m_vec, jnp.float32(0.0)))

        _, sum_idx = extract(maxbuf, m_all, posidx, None, True)
        _, sum_m = extract(minbuf, mn_all, negidx, negval, False)

        pos_tmp = _ALPHA * (s_pos - sum_idx)
        neg_tmp = _ALPHA * (sum_m - s_negsum)

        for t in range(2):
            pidx = posidx[pl.ds(t * _L, _L)]
            plsc.store_scatter(out_v, [pidx],
                               pidx.astype(jnp.float32) + pos_tmp)
            nidx = negidx[pl.ds(t * _L, _L)]
            nval = negval[pl.ds(t * _L, _L)]
            plsc.store_scatter(out_v, [nidx], nval - neg_tmp)

        pltpu.sync_copy(out_v, o_hbm.at[row])

        for t in range(2):
            pidx = posidx[pl.ds(t * _L, _L)]
            plsc.store_scatter(out_v, [pidx], jnp.zeros((_L,), jnp.float32))
            nidx = negidx[pl.ds(t * _L, _L)]
            plsc.store_scatter(out_v, [nidx], jnp.zeros((_L,), jnp.float32))


def kernel(x):
    mesh = plsc.VectorSubcoreMesh(core_axis_name="c", subcore_axis_name="s",
                                  num_cores=2, num_subcores=16)
    f = pl.kernel(
        _sc_body,
        out_type=jax.ShapeDtypeStruct((_B, _D), jnp.float32),
        mesh=mesh,
        compiler_params=pltpu.CompilerParams(use_tc_tiling_on_sc=False,
                                             needs_layout_passes=False),
        scratch_types=[
            pltpu.VMEM((_D,), jnp.float32),
            pltpu.VMEM((_D,), jnp.float32),
            pltpu.VMEM((_GROUPS * _L,), jnp.float32),
            pltpu.VMEM((_GROUPS * _L,), jnp.float32),
            pltpu.VMEM((32,), jnp.int32),
            pltpu.VMEM((32,), jnp.int32),
            pltpu.VMEM((32,), jnp.float32),
        ],
    )
    return f(x)


# speedup vs baseline: 11.5018x; 1.1854x over previous
import jax
import jax.numpy as jnp
from jax import lax
from jax.experimental import pallas as pl
from jax.experimental.pallas import tpu as pltpu
from jax.experimental.pallas import tpu_sc as plsc

_ALPHA = 6.26
_D = 32768
_B = 128
_L = 16
_GROUPS = 128
_VPG = 16
_NSUB = 32
_ROWS_PER_W = _B // _NSUB
_GV = _GROUPS // _L


def _iota16():
    return lax.broadcasted_iota(jnp.int32, (_L,), 0)


def _splat_i32(x):
    return jnp.broadcast_to(x, (_L,)).astype(jnp.int32)


def _splat_f32(x):
    return jnp.broadcast_to(x, (_L,)).astype(jnp.float32)


def _store1(ref, pos, val, i16):
    plsc.store_scatter(ref, [_splat_i32(pos)], val, mask=i16 == 0)


def _sc_body(x_hbm, o_hbm, row_a, row_b, out_v, maxbuf, minbuf,
             posidx, negidx, negval, sem_a, sem_b):
    i16 = _iota16()
    wid = lax.axis_index("c") * 16 + lax.axis_index("s")
    rows = (row_a, row_b)
    sems = (sem_a, sem_b)

    def zero_step(i, _):
        out_v[pl.ds(i * _L, _L)] = jnp.zeros((_L,), jnp.float32)
        return 0

    copies = [pltpu.async_copy(x_hbm.at[wid * _ROWS_PER_W], row_a, sem_a)]
    lax.fori_loop(0, _D // _L, zero_step, 0)

    for j in range(_ROWS_PER_W):
        row = wid * _ROWS_PER_W + j
        row_v = rows[j % 2]
        copies.pop().wait()
        if j + 1 < _ROWS_PER_W:
            copies.append(pltpu.async_copy(
                x_hbm.at[row + 1], rows[(j + 1) % 2], sems[(j + 1) % 2]))

        def p1_group(g, carry):
            s_all, s_abs, m_all, mn_all = carry
            base = g * (_VPG * _L)
            v = row_v[pl.ds(base, _L)]
            gmx = v
            gmn = v
            sa = v
            sb = jnp.abs(v)
            for k in range(1, _VPG):
                v = row_v[pl.ds(base + k * _L, _L)]
                gmx = jnp.maximum(gmx, v)
                gmn = jnp.minimum(gmn, v)
                sa = sa + v
                sb = sb + jnp.abs(v)
            plsc.store_scatter(maxbuf, [i16 * _GROUPS + g], gmx)
            plsc.store_scatter(minbuf, [i16 * _GROUPS + g], gmn)
            return (s_all + sa, s_abs + sb,
                    jnp.maximum(m_all, gmx), jnp.minimum(mn_all, gmn))

        zv = jnp.zeros((_L,), jnp.float32)
        s_all, s_abs, m_all, mn_all = lax.fori_loop(
            0, _GROUPS, p1_group,
            (zv, zv, jnp.full((_L,), -jnp.inf, jnp.float32),
             jnp.full((_L,), jnp.inf, jnp.float32)))
        sum_all = jnp.sum(s_all)
        sum_abs = jnp.sum(s_abs)
        s_pos = 0.5 * (sum_abs + sum_all)
        s_negsum = 0.5 * (sum_all - sum_abs)

        def extract(buf, m_vec, idx_list, val_list, is_max):
            red = jnp.max if is_max else jnp.min

            def step(i, carry):
                m_vec, acc = carry
                m = red(m_vec)
                l_v = plsc.all_reduce_ffs(m_vec == m)
                found = jnp.full((_L,), _GROUPS, jnp.int32)
                gvs = []
                for t in range(_GV):
                    gvec = plsc.load_gather(
                        buf, [l_v * _GROUPS + (t * _L) + i16])
                    gvs.append(gvec)
                    found = jnp.minimum(
                        found, jnp.where(gvec == m, i16 + t * _L, _GROUPS))
                g_v = _splat_i32(jnp.min(found))
                base_v = g_v * (_VPG * _L) + l_v
                didx = base_v + i16 * _L
                v = plsc.load_gather(row_v, [didx])
                kl_v = plsc.all_reduce_ffs(v == m)
                elem_v = base_v + kl_v * _L
                v2 = jnp.where(i16 == kl_v, 0.0, v)
                plsc.store_scatter(row_v, [didx], v2)
                newm = _splat_f32(red(v2))
                _store1(buf, l_v * _GROUPS + g_v, newm, i16)
                _store1(idx_list, i, elem_v, i16)
                if val_list is not None:
                    _store1(val_list, i, _splat_f32(m), i16)
                lacc = None
                for t in range(_GV):
                    gm = jnp.where(t * _L + i16 == g_v, newm, gvs[t])
                    lacc = gm if lacc is None else (
                        jnp.maximum(lacc, gm) if is_max
                        else jnp.minimum(lacc, gm))
                ml = red(lacc)
                m_vec = jnp.where(i16 == l_v, ml, m_vec)
                acc = acc + (elem_v.astype(jnp.float32) if is_max
                             else _splat_f32(m))
                return (m_vec, acc)

            m_vec, acc = lax.fori_loop(0, 32, step, (m_vec, zv))
            return jnp.max(acc)

        sum_idx = extract(maxbuf, m_all, posidx, None, True)
        sum_m = extract(minbuf, mn_all, negidx, negval, False)

        pos_tmp = _ALPHA * (s_pos - sum_idx)
        neg_tmp = _ALPHA * (sum_m - s_negsum)

        for t in range(2):
            pidx = posidx[pl.ds(t * _L, _L)]
            plsc.store_scatter(out_v, [pidx],
                               pidx.astype(jnp.float32) + pos_tmp)
            nidx = negidx[pl.ds(t * _L, _L)]
            nval = negval[pl.ds(t * _L, _L)]
            plsc.store_scatter(out_v, [nidx], nval - neg_tmp)

        pltpu.sync_copy(out_v, o_hbm.at[row])

        for t in range(2):
            pidx = posidx[pl.ds(t * _L, _L)]
            plsc.store_scatter(out_v, [pidx], jnp.zeros((_L,), jnp.float32))
            nidx = negidx[pl.ds(t * _L, _L)]
            plsc.store_scatter(out_v, [nidx], jnp.zeros((_L,), jnp.float32))


def kernel(x):
    mesh = plsc.VectorSubcoreMesh(core_axis_name="c", subcore_axis_name="s",
                                  num_cores=2, num_subcores=16)
    f = pl.kernel(
        _sc_body,
        out_type=jax.ShapeDtypeStruct((_B, _D), jnp.float32),
        mesh=mesh,
        compiler_params=pltpu.CompilerParams(use_tc_tiling_on_sc=False,
                                             needs_layout_passes=False),
        scratch_types=[
            pltpu.VMEM((_D,), jnp.float32),
            pltpu.VMEM((_D,), jnp.float32),
            pltpu.VMEM((_D,), jnp.float32),
            pltpu.VMEM((_GROUPS * _L,), jnp.float32),
            pltpu.VMEM((_GROUPS * _L,), jnp.float32),
            pltpu.VMEM((32,), jnp.int32),
            pltpu.VMEM((32,), jnp.int32),
            pltpu.VMEM((32,), jnp.float32),
            pltpu.SemaphoreType.DMA,
            pltpu.SemaphoreType.DMA,
        ],
    )
    return f(x)


# speedup vs baseline: 17.8795x; 1.5545x over previous
import jax
import jax.numpy as jnp
from jax import lax
from jax.experimental import pallas as pl
from jax.experimental.pallas import tpu as pltpu
from jax.experimental.pallas import tpu_sc as plsc

_ALPHA = 6.26
_D = 32768
_B = 128
_L = 16
_GROUPS = 128
_VPG = 16
_NSUB = 32
_ROWS_PER_W = _B // _NSUB
_GV = _GROUPS // _L


def _iota16():
    return lax.broadcasted_iota(jnp.int32, (_L,), 0)


def _splat_i32(x):
    return jnp.broadcast_to(x, (_L,)).astype(jnp.int32)


def _splat_f32(x):
    return jnp.broadcast_to(x, (_L,)).astype(jnp.float32)


def _store1(ref, pos, val, i16):
    plsc.store_scatter(ref, [_splat_i32(pos)], val, mask=i16 == 0)


def _sc_body(x_hbm, o_hbm, row_a, row_b, out_v, maxbuf, minbuf,
             posidx, negidx, negval, sem_a, sem_b):
    i16 = _iota16()
    wid = lax.axis_index("c") * 16 + lax.axis_index("s")
    rows = (row_a, row_b)
    sems = (sem_a, sem_b)

    def zero_step(i, _):
        out_v[pl.ds(i * _L, _L)] = jnp.zeros((_L,), jnp.float32)
        return 0

    copies = [pltpu.async_copy(x_hbm.at[wid * _ROWS_PER_W], row_a, sem_a)]
    lax.fori_loop(0, _D // _L, zero_step, 0)

    for j in range(_ROWS_PER_W):
        row = wid * _ROWS_PER_W + j
        row_v = rows[j % 2]
        copies.pop().wait()
        if j + 1 < _ROWS_PER_W:
            copies.append(pltpu.async_copy(
                x_hbm.at[row + 1], rows[(j + 1) % 2], sems[(j + 1) % 2]))

        def p1_group(g, carry):
            s_all, s_abs, m_all, mn_all = carry
            base = g * (_VPG * _L)
            v = row_v[pl.ds(base, _L)]
            gmx = v
            gmn = v
            sa = v
            sb = jnp.abs(v)
            for k in range(1, _VPG):
                v = row_v[pl.ds(base + k * _L, _L)]
                gmx = jnp.maximum(gmx, v)
                gmn = jnp.minimum(gmn, v)
                sa = sa + v
                sb = sb + jnp.abs(v)
            plsc.store_scatter(maxbuf, [i16 * _GROUPS + g], gmx)
            plsc.store_scatter(minbuf, [i16 * _GROUPS + g], gmn)
            return (s_all + sa, s_abs + sb,
                    jnp.maximum(m_all, gmx), jnp.minimum(mn_all, gmn))

        zv = jnp.zeros((_L,), jnp.float32)
        s_all, s_abs, m_all, mn_all = lax.fori_loop(
            0, _GROUPS, p1_group,
            (zv, zv, jnp.full((_L,), -jnp.inf, jnp.float32),
             jnp.full((_L,), jnp.inf, jnp.float32)))
        sum_all = jnp.sum(s_all)
        sum_abs = jnp.sum(s_abs)
        s_pos = 0.5 * (sum_abs + sum_all)
        s_negsum = 0.5 * (sum_all - sum_abs)

        def extract(buf, m_vec, idx_list, val_list, is_max):
            red = jnp.max if is_max else jnp.min

            def step(i, carry):
                m_vec, acc = carry
                m = red(m_vec)
                l_v = plsc.all_reduce_ffs(m_vec == m)
                found = jnp.full((_L,), _GROUPS, jnp.int32)
                gvs = []
                for t in range(_GV):
                    gvec = plsc.load_gather(
                        buf, [l_v * _GROUPS + (t * _L) + i16])
                    gvs.append(gvec)
                    found = jnp.minimum(
                        found, jnp.where(gvec == m, i16 + t * _L, _GROUPS))
                g_v = _splat_i32(jnp.min(found))
                base_v = g_v * (_VPG * _L) + l_v
                didx = base_v + i16 * _L
                v = plsc.load_gather(row_v, [didx])
                kl_v = plsc.all_reduce_ffs(v == m)
                elem_v = base_v + kl_v * _L
                v2 = jnp.where(i16 == kl_v, 0.0, v)
                plsc.store_scatter(row_v, [didx], v2)
                newm = _splat_f32(red(v2))
                _store1(buf, l_v * _GROUPS + g_v, newm, i16)
                _store1(idx_list, i, elem_v, i16)
                if val_list is not None:
                    _store1(val_list, i, _splat_f32(m), i16)
                lacc = None
                for t in range(_GV):
                    gm = jnp.where(t * _L + i16 == g_v, newm, gvs[t])
                    lacc = gm if lacc is None else (
                        jnp.maximum(lacc, gm) if is_max
                        else jnp.minimum(lacc, gm))
                ml = red(lacc)
                m_vec = jnp.where(i16 == l_v, ml, m_vec)
                acc = acc + (elem_v.astype(jnp.float32) if is_max
                             else _splat_f32(m))
                return (m_vec, acc)

            m_vec, acc = lax.fori_loop(0, 32, step, (m_vec, zv))
            return jnp.max(acc)

        sum_idx = extract(maxbuf, m_all, posidx, None, True)
        sum_m = extract(minbuf, mn_all, negidx, negval, False)

        pos_tmp = _ALPHA * (s_pos - sum_idx)
        neg_tmp = _ALPHA * (sum_m - s_negsum)

        for t in range(2):
            pidx = posidx[pl.ds(t * _L, _L)]
            plsc.store_scatter(out_v, [pidx],
                               pidx.astype(jnp.float32) + pos_tmp)
            nidx = negidx[pl.ds(t * _L, _L)]
            nval = negval[pl.ds(t * _L, _L)]
            plsc.store_scatter(out_v, [nidx], nval - neg_tmp)

        pltpu.sync_copy(out_v, o_hbm.at[row])

        for t in range(2):
            pidx = posidx[pl.ds(t * _L, _L)]
            plsc.store_scatter(out_v, [pidx], jnp.zeros((_L,), jnp.float32))
            nidx = negidx[pl.ds(t * _L, _L)]
            plsc.store_scatter(out_v, [nidx], jnp.zeros((_L,), jnp.float32))


def kernel(x):
    mesh = plsc.VectorSubcoreMesh(core_axis_name="c", subcore_axis_name="s",
                                  num_cores=2, num_subcores=16)
    f = pl.kernel(
        _sc_body,
        out_type=jax.ShapeDtypeStruct((_B, _D), jnp.float32),
        mesh=mesh,
        compiler_params=pltpu.CompilerParams(use_tc_tiling_on_sc=True,
                                             needs_layout_passes=False),
        scratch_types=[
            pltpu.VMEM((_D,), jnp.float32),
            pltpu.VMEM((_D,), jnp.float32),
            pltpu.VMEM((_D,), jnp.float32),
            pltpu.VMEM((_GROUPS * _L,), jnp.float32),
            pltpu.VMEM((_GROUPS * _L,), jnp.float32),
            pltpu.VMEM((32,), jnp.int32),
            pltpu.VMEM((32,), jnp.int32),
            pltpu.VMEM((32,), jnp.float32),
            pltpu.SemaphoreType.DMA,
            pltpu.SemaphoreType.DMA,
        ],
    )
    return f(x)


# speedup vs baseline: 22.7263x; 1.2711x over previous
import jax
import jax.numpy as jnp
from jax import lax
from jax.experimental import pallas as pl
from jax.experimental.pallas import tpu as pltpu
from jax.experimental.pallas import tpu_sc as plsc

_ALPHA = 6.26
_D = 32768
_B = 128
_L = 16
_GROUPS = 128
_VPG = 16
_NSUB = 32
_ROWS_PER_W = _B // _NSUB
_GV = _GROUPS // _L


def _iota16():
    return lax.broadcasted_iota(jnp.int32, (_L,), 0)


def _splat_i32(x):
    return jnp.broadcast_to(x, (_L,)).astype(jnp.int32)


def _splat_f32(x):
    return jnp.broadcast_to(x, (_L,)).astype(jnp.float32)


def _store1(ref, pos, val, i16):
    plsc.store_scatter(ref, [_splat_i32(pos)], val, mask=i16 == 0)


def _sc_body(x_hbm, o_hbm, row_a, row_b, out_v, maxbuf, minbuf,
             pos_a, pos_b, neg_a, neg_b, nval_a, nval_b,
             sem_a, sem_b, osem):
    i16 = _iota16()
    wid = lax.axis_index("c") * 16 + lax.axis_index("s")
    rows = (row_a, row_b)
    plists = (pos_a, pos_b)
    nlists = (neg_a, neg_b)
    vlists = (nval_a, nval_b)
    sems = (sem_a, sem_b)

    def zero_block(i, _):
        for k in range(_VPG):
            out_v[pl.ds(i * (_VPG * _L) + k * _L, _L)] = \
                jnp.zeros((_L,), jnp.float32)
        return 0

    in_copies = [pltpu.async_copy(x_hbm.at[wid * _ROWS_PER_W], row_a, sem_a)]
    lax.fori_loop(0, _GROUPS, zero_block, 0)

    out_copy = [None]

    for j in range(_ROWS_PER_W):
        row = wid * _ROWS_PER_W + j
        row_v = rows[j % 2]
        posidx = plists[j % 2]
        negidx = nlists[j % 2]
        negval = vlists[j % 2]
        in_copies.pop().wait()
        if j + 1 < _ROWS_PER_W:
            in_copies.append(pltpu.async_copy(
                x_hbm.at[row + 1], rows[(j + 1) % 2], sems[(j + 1) % 2]))

        def p1_group(g, carry):
            s_all, s_abs, m_all, mn_all = carry
            base = g * (_VPG * _L)
            v = row_v[pl.ds(base, _L)]
            gmx = v
            gmn = v
            sa = v
            sb = jnp.abs(v)
            for k in range(1, _VPG):
                v = row_v[pl.ds(base + k * _L, _L)]
                gmx = jnp.maximum(gmx, v)
                gmn = jnp.minimum(gmn, v)
                sa = sa + v
                sb = sb + jnp.abs(v)
            plsc.store_scatter(maxbuf, [i16 * _GROUPS + g], gmx)
            plsc.store_scatter(minbuf, [i16 * _GROUPS + g], gmn)
            return (s_all + sa, s_abs + sb,
                    jnp.maximum(m_all, gmx), jnp.minimum(mn_all, gmn))

        zv = jnp.zeros((_L,), jnp.float32)
        s_all, s_abs, m_all, mn_all = lax.fori_loop(
            0, _GROUPS, p1_group,
            (zv, zv, jnp.full((_L,), -jnp.inf, jnp.float32),
             jnp.full((_L,), jnp.inf, jnp.float32)))
        sum_all = jnp.sum(s_all)
        sum_abs = jnp.sum(s_abs)
        s_pos = 0.5 * (sum_abs + sum_all)
        s_negsum = 0.5 * (sum_all - sum_abs)

        def side(buf, m_vec, idx_list, val_list, is_max, i, acc):
            red = jnp.max if is_max else jnp.min
            m = red(m_vec)
            l_v = plsc.all_reduce_ffs(m_vec == m)
            found = jnp.full((_L,), _GROUPS, jnp.int32)
            gvs = []
            for t in range(_GV):
                gvec = plsc.load_gather(
                    buf, [l_v * _GROUPS + (t * _L) + i16])
                gvs.append(gvec)
                found = jnp.minimum(
                    found, jnp.where(gvec == m, i16 + t * _L, _GROUPS))
            g_v = _splat_i32(jnp.min(found))
            base_v = g_v * (_VPG * _L) + l_v
            didx = base_v + i16 * _L
            v = plsc.load_gather(row_v, [didx])
            kl_v = plsc.all_reduce_ffs(v == m)
            elem_v = base_v + kl_v * _L
            v2 = jnp.where(i16 == kl_v, 0.0, v)
            plsc.store_scatter(row_v, [didx], v2)
            newm = _splat_f32(red(v2))
            _store1(buf, l_v * _GROUPS + g_v, newm, i16)
            _store1(idx_list, i, elem_v, i16)
            if val_list is not None:
                _store1(val_list, i, _splat_f32(m), i16)
            lacc = None
            for t in range(_GV):
                gm = jnp.where(t * _L + i16 == g_v, newm, gvs[t])
                lacc = gm if lacc is None else (
                    jnp.maximum(lacc, gm) if is_max
                    else jnp.minimum(lacc, gm))
            ml = red(lacc)
            m_vec = jnp.where(i16 == l_v, ml, m_vec)
            acc = acc + (elem_v.astype(jnp.float32) if is_max
                         else _splat_f32(m))
            return m_vec, acc

        def step(i, carry):
            mx_vec, accp, mn_vec, accn = carry
            mx_vec, accp = side(maxbuf, mx_vec, posidx, None, True, i, accp)
            mn_vec, accn = side(minbuf, mn_vec, negidx, negval, False, i,
                                accn)
            return (mx_vec, accp, mn_vec, accn)

        _, accp, _, accn = lax.fori_loop(
            0, 32, step, (m_all, zv, mn_all, zv))
        sum_idx = jnp.max(accp)
        sum_m = jnp.min(accn)

        pos_tmp = _ALPHA * (s_pos - sum_idx)
        neg_tmp = _ALPHA * (sum_m - s_negsum)

        if out_copy[0] is not None:
            out_copy[0].wait()
            oidx = plists[(j + 1) % 2]
            onidx = nlists[(j + 1) % 2]
            for t in range(2):
                opi = oidx[pl.ds(t * _L, _L)]
                plsc.store_scatter(out_v, [opi],
                                   jnp.zeros((_L,), jnp.float32))
                oni = onidx[pl.ds(t * _L, _L)]
                plsc.store_scatter(out_v, [oni],
                                   jnp.zeros((_L,), jnp.float32))

        for t in range(2):
            pidx = posidx[pl.ds(t * _L, _L)]
            plsc.store_scatter(out_v, [pidx],
                               pidx.astype(jnp.float32) + pos_tmp)
            nidx = negidx[pl.ds(t * _L, _L)]
            nval = negval[pl.ds(t * _L, _L)]
            plsc.store_scatter(out_v, [nidx], nval - neg_tmp)

        out_copy[0] = pltpu.async_copy(out_v, o_hbm.at[row], osem)

    out_copy[0].wait()


def kernel(x):
    mesh = plsc.VectorSubcoreMesh(core_axis_name="c", subcore_axis_name="s",
                                  num_cores=2, num_subcores=16)
    f = pl.kernel(
        _sc_body,
        out_type=jax.ShapeDtypeStruct((_B, _D), jnp.float32),
        mesh=mesh,
        compiler_params=pltpu.CompilerParams(use_tc_tiling_on_sc=True,
                                             needs_layout_passes=False),
        scratch_types=[
            pltpu.VMEM((_D,), jnp.float32),
            pltpu.VMEM((_D,), jnp.float32),
            pltpu.VMEM((_D,), jnp.float32),
            pltpu.VMEM((_GROUPS * _L,), jnp.float32),
            pltpu.VMEM((_GROUPS * _L,), jnp.float32),
            pltpu.VMEM((32,), jnp.int32),
            pltpu.VMEM((32,), jnp.int32),
            pltpu.VMEM((32,), jnp.int32),
            pltpu.VMEM((32,), jnp.int32),
            pltpu.VMEM((32,), jnp.float32),
            pltpu.VMEM((32,), jnp.float32),
            pltpu.SemaphoreType.DMA,
            pltpu.SemaphoreType.DMA,
            pltpu.SemaphoreType.DMA,
        ],
    )
    return f(x)
